# initial kernel scaffold (unmeasured)
import jax
import jax.numpy as jnp
from jax import lax
from jax.experimental import pallas as pl
from jax.experimental.pallas import tpu as pltpu


def kernel(
    u,
):
    def body(*refs):
        pass

    out_shape = jax.ShapeDtypeStruct(..., jnp.float32)
    return pl.pallas_call(body, out_shape=out_shape)(...)


# baseline (device time: 39941 ns/iter reference)
import jax
import jax.numpy as jnp
from jax import lax
from jax.experimental import pallas as pl
from jax.experimental.pallas import tpu as pltpu

SX, SY, SZ = 2, 4, 4


def kernel(u):
    n0, n1, n2 = u.shape

    def body(u_ref, out_ref, hxm, hxp, hym, hyp, hzm, hzp,
             sxm, sxp, sym, syp, szm, szp, send_sems, recv_sems):
        mx = lax.axis_index("x")
        my = lax.axis_index("y")
        mz = lax.axis_index("z")

        halos = (hxm, hxp, hym, hyp, hzm, hzp)
        stage = (sxm, sxp, sym, syp, szm, szp)

        sxm[:, :, :] = u_ref[pl.ds(0, 1), :, :]
        sxp[:, :, :] = u_ref[pl.ds(n0 - 1, 1), :, :]
        sym[:, :, :] = u_ref[:, pl.ds(0, 1), :]
        syp[:, :, :] = u_ref[:, pl.ds(n1 - 1, 1), :]
        szm[:, :, :] = u_ref[:, :, pl.ds(0, 1)]
        szp[:, :, :] = u_ref[:, :, pl.ds(n2 - 1, 1)]

        has = (
            mx > 0, mx < SX - 1,
            my > 0, my < SY - 1,
            mz > 0, mz < SZ - 1,
        )

        def nbr_id(axis, delta):
            return (
                mx + (delta if axis == 0 else 0),
                my + (delta if axis == 1 else 0),
                mz + (delta if axis == 2 else 0),
            )

        for d in range(6):
            axis, plus = d // 2, d % 2
            delta = 1 if plus else -1
            dst_idx = 2 * axis + (0 if plus else 1)

            @pl.when(has[d])
            def _(d=d, axis=axis, delta=delta, dst_idx=dst_idx):
                rdma = pltpu.make_async_remote_copy(
                    src_ref=stage[d],
                    dst_ref=halos[dst_idx],
                    send_sem=send_sems.at[d],
                    recv_sem=recv_sems.at[dst_idx],
                    device_id=nbr_id(axis, delta),
                    device_id_type=pl.DeviceIdType.MESH,
                )
                rdma.start()

        for h in range(6):

            @pl.when(has[h])
            def _(h=h):
                rdma = pltpu.make_async_remote_copy(
                    src_ref=stage[h],
                    dst_ref=halos[h],
                    send_sem=send_sems.at[h],
                    recv_sem=recv_sems.at[h],
                    device_id=(mx, my, mz),
                    device_id_type=pl.DeviceIdType.MESH,
                )
                rdma.wait_recv()

        x = u_ref[:, :, :]
        upx = jnp.concatenate([hxm[:, :, :], x[:-1]], axis=0)
        dnx = jnp.concatenate([x[1:], hxp[:, :, :]], axis=0)
        upy = jnp.concatenate([hym[:, :, :], x[:, :-1]], axis=1)
        dny = jnp.concatenate([x[:, 1:], hyp[:, :, :]], axis=1)
        upz = jnp.concatenate([hzm[:, :, :], x[:, :, :-1]], axis=2)
        dnz = jnp.concatenate([x[:, :, 1:], hzp[:, :, :]], axis=2)
        v = upx + dnx + upy + dny + upz + dnz - 6.0 * x

        ii = lax.broadcasted_iota(jnp.int32, x.shape, 0)
        jj = lax.broadcasted_iota(jnp.int32, x.shape, 1)
        kk = lax.broadcasted_iota(jnp.int32, x.shape, 2)
        boundary = (
            ((mx == 0) & (ii == 0)) | ((mx == SX - 1) & (ii == n0 - 1))
            | ((my == 0) & (jj == 0)) | ((my == SY - 1) & (jj == n1 - 1))
            | ((mz == 0) & (kk == 0)) | ((mz == SZ - 1) & (kk == n2 - 1))
        )
        out_ref[:, :, :] = jnp.where(boundary, 0.0, v)

        for d in range(6):
            axis, plus = d // 2, d % 2
            delta = 1 if plus else -1
            dst_idx = 2 * axis + (0 if plus else 1)

            @pl.when(has[d])
            def _(d=d, axis=axis, delta=delta, dst_idx=dst_idx):
                rdma = pltpu.make_async_remote_copy(
                    src_ref=stage[d],
                    dst_ref=halos[dst_idx],
                    send_sem=send_sems.at[d],
                    recv_sem=recv_sems.at[dst_idx],
                    device_id=nbr_id(axis, delta),
                    device_id_type=pl.DeviceIdType.MESH,
                )
                rdma.wait_send()

    return pl.pallas_call(
        body,
        out_shape=jax.ShapeDtypeStruct((n0, n1, n2), u.dtype),
        in_specs=[pl.BlockSpec(memory_space=pltpu.VMEM)],
        out_specs=pl.BlockSpec(memory_space=pltpu.VMEM),
        scratch_shapes=[
            pltpu.VMEM((1, n1, n2), u.dtype),
            pltpu.VMEM((1, n1, n2), u.dtype),
            pltpu.VMEM((n0, 1, n2), u.dtype),
            pltpu.VMEM((n0, 1, n2), u.dtype),
            pltpu.VMEM((n0, n1, 1), u.dtype),
            pltpu.VMEM((n0, n1, 1), u.dtype),
            pltpu.VMEM((1, n1, n2), u.dtype),
            pltpu.VMEM((1, n1, n2), u.dtype),
            pltpu.VMEM((n0, 1, n2), u.dtype),
            pltpu.VMEM((n0, 1, n2), u.dtype),
            pltpu.VMEM((n0, n1, 1), u.dtype),
            pltpu.VMEM((n0, n1, 1), u.dtype),
            pltpu.SemaphoreType.DMA((6,)),
            pltpu.SemaphoreType.DMA((6,)),
        ],
    )(u)


# device time: 32782 ns/iter; 1.2184x vs baseline; 1.2184x over previous
import jax
import jax.numpy as jnp
from jax import lax
from jax.experimental import pallas as pl
from jax.experimental.pallas import tpu as pltpu

SX, SY, SZ = 2, 4, 4


def kernel(u):
    n0, n1, n2 = u.shape

    def body(u_ref, out_ref, hxm, hxp, hym, hyp, hzm, hzp,
             sxm, sxp, sym, syp, szm, szp, send_sems, recv_sems):
        mx = lax.axis_index("x")
        my = lax.axis_index("y")
        mz = lax.axis_index("z")

        halos = (hxm, hxp, hym, hyp, hzm, hzp)
        stage = (sxm, sxp, sym, syp, szm, szp)

        sxm[:, :, :] = u_ref[pl.ds(0, 1), :, :]
        sxp[:, :, :] = u_ref[pl.ds(n0 - 1, 1), :, :]
        sym[:, :, :] = u_ref[:, pl.ds(0, 1), :]
        syp[:, :, :] = u_ref[:, pl.ds(n1 - 1, 1), :]
        szm[:, :, :] = u_ref[:, :, pl.ds(0, 1)]
        szp[:, :, :] = u_ref[:, :, pl.ds(n2 - 1, 1)]

        has = (
            mx > 0, mx < SX - 1,
            my > 0, my < SY - 1,
            mz > 0, mz < SZ - 1,
        )

        def nbr_id(axis, delta):
            return (
                mx + (delta if axis == 0 else 0),
                my + (delta if axis == 1 else 0),
                mz + (delta if axis == 2 else 0),
            )

        barrier_sem = pltpu.get_barrier_semaphore()
        n_nbrs = jnp.int32(0)
        for d in range(6):
            axis, plus = d // 2, d % 2
            delta = 1 if plus else -1
            n_nbrs = n_nbrs + has[d].astype(jnp.int32)

            @pl.when(has[d])
            def _(axis=axis, delta=delta):
                pl.semaphore_signal(
                    barrier_sem, inc=1,
                    device_id=nbr_id(axis, delta),
                    device_id_type=pl.DeviceIdType.MESH,
                )
        pl.semaphore_wait(barrier_sem, n_nbrs)

        for d in range(6):
            axis, plus = d // 2, d % 2
            delta = 1 if plus else -1
            dst_idx = 2 * axis + (0 if plus else 1)

            @pl.when(has[d])
            def _(d=d, axis=axis, delta=delta, dst_idx=dst_idx):
                rdma = pltpu.make_async_remote_copy(
                    src_ref=stage[d],
                    dst_ref=halos[dst_idx],
                    send_sem=send_sems.at[d],
                    recv_sem=recv_sems.at[dst_idx],
                    device_id=nbr_id(axis, delta),
                    device_id_type=pl.DeviceIdType.MESH,
                )
                rdma.start()

        for h in range(6):

            @pl.when(has[h])
            def _(h=h):
                rdma = pltpu.make_async_remote_copy(
                    src_ref=stage[h],
                    dst_ref=halos[h],
                    send_sem=send_sems.at[h],
                    recv_sem=recv_sems.at[h],
                    device_id=(mx, my, mz),
                    device_id_type=pl.DeviceIdType.MESH,
                )
                rdma.wait_recv()

        x = u_ref[:, :, :]
        upx = jnp.concatenate([hxm[:, :, :], x[:-1]], axis=0)
        dnx = jnp.concatenate([x[1:], hxp[:, :, :]], axis=0)
        upy = jnp.concatenate([hym[:, :, :], x[:, :-1]], axis=1)
        dny = jnp.concatenate([x[:, 1:], hyp[:, :, :]], axis=1)
        upz = jnp.concatenate([hzm[:, :, :], x[:, :, :-1]], axis=2)
        dnz = jnp.concatenate([x[:, :, 1:], hzp[:, :, :]], axis=2)
        v = upx + dnx + upy + dny + upz + dnz - 6.0 * x

        ii = lax.broadcasted_iota(jnp.int32, x.shape, 0)
        jj = lax.broadcasted_iota(jnp.int32, x.shape, 1)
        kk = lax.broadcasted_iota(jnp.int32, x.shape, 2)
        boundary = (
            ((mx == 0) & (ii == 0)) | ((mx == SX - 1) & (ii == n0 - 1))
            | ((my == 0) & (jj == 0)) | ((my == SY - 1) & (jj == n1 - 1))
            | ((mz == 0) & (kk == 0)) | ((mz == SZ - 1) & (kk == n2 - 1))
        )
        out_ref[:, :, :] = jnp.where(boundary, 0.0, v)

        for d in range(6):
            axis, plus = d // 2, d % 2
            delta = 1 if plus else -1
            dst_idx = 2 * axis + (0 if plus else 1)

            @pl.when(has[d])
            def _(d=d, axis=axis, delta=delta, dst_idx=dst_idx):
                rdma = pltpu.make_async_remote_copy(
                    src_ref=stage[d],
                    dst_ref=halos[dst_idx],
                    send_sem=send_sems.at[d],
                    recv_sem=recv_sems.at[dst_idx],
                    device_id=nbr_id(axis, delta),
                    device_id_type=pl.DeviceIdType.MESH,
                )
                rdma.wait_send()

    return pl.pallas_call(
        body,
        out_shape=jax.ShapeDtypeStruct((n0, n1, n2), u.dtype),
        in_specs=[pl.BlockSpec(memory_space=pltpu.VMEM)],
        out_specs=pl.BlockSpec(memory_space=pltpu.VMEM),
        scratch_shapes=[
            pltpu.VMEM((1, n1, n2), u.dtype),
            pltpu.VMEM((1, n1, n2), u.dtype),
            pltpu.VMEM((n0, 1, n2), u.dtype),
            pltpu.VMEM((n0, 1, n2), u.dtype),
            pltpu.VMEM((n0, n1, 1), u.dtype),
            pltpu.VMEM((n0, n1, 1), u.dtype),
            pltpu.VMEM((1, n1, n2), u.dtype),
            pltpu.VMEM((1, n1, n2), u.dtype),
            pltpu.VMEM((n0, 1, n2), u.dtype),
            pltpu.VMEM((n0, 1, n2), u.dtype),
            pltpu.VMEM((n0, n1, 1), u.dtype),
            pltpu.VMEM((n0, n1, 1), u.dtype),
            pltpu.SemaphoreType.DMA((6,)),
            pltpu.SemaphoreType.DMA((6,)),
        ],
        compiler_params=pltpu.CompilerParams(collective_id=0),
    )(u)


# device time: 10261 ns/iter; 3.8925x vs baseline; 3.1948x over previous
import jax
import jax.numpy as jnp
from jax import lax
from jax.experimental import pallas as pl
from jax.experimental.pallas import tpu as pltpu

SX, SY, SZ = 2, 4, 4


def kernel(u):
    n0, n1, n2 = u.shape

    def body(u_ref, out_ref, hxm, hxp, hym, hyp, hzm, hzp,
             sxm, sxp, sym, syp, szm, szp, send_sems, recv_sems):
        mx = lax.axis_index("x")
        my = lax.axis_index("y")
        mz = lax.axis_index("z")

        halos = (hxm, hxp, hym, hyp, hzm, hzp)
        stage = (sxm, sxp, sym, syp, szm, szp)

        sxm[0, :, :] = u_ref[0, :, :]
        sxp[0, :, :] = u_ref[n0 - 1, :, :]
        sym[0, :, :] = u_ref[:, 0, :]
        syp[0, :, :] = u_ref[:, n1 - 1, :]
        szm[0, :, :] = u_ref[:, :, 0]
        szp[0, :, :] = u_ref[:, :, n2 - 1]

        has = (
            mx > 0, mx < SX - 1,
            my > 0, my < SY - 1,
            mz > 0, mz < SZ - 1,
        )

        def nbr_id(axis, delta):
            return (
                mx + (delta if axis == 0 else 0),
                my + (delta if axis == 1 else 0),
                mz + (delta if axis == 2 else 0),
            )

        barrier_sem = pltpu.get_barrier_semaphore()
        n_nbrs = jnp.int32(0)
        for d in range(6):
            axis, plus = d // 2, d % 2
            delta = 1 if plus else -1
            n_nbrs = n_nbrs + has[d].astype(jnp.int32)

            @pl.when(has[d])
            def _(axis=axis, delta=delta):
                pl.semaphore_signal(
                    barrier_sem, inc=1,
                    device_id=nbr_id(axis, delta),
                    device_id_type=pl.DeviceIdType.MESH,
                )


        x = u_ref[:, :, :]
        zx = jnp.zeros((1, n1, n2), x.dtype)
        zy = jnp.zeros((n0, 1, n2), x.dtype)
        zz = jnp.zeros((n0, n1, 1), x.dtype)
        upx = jnp.concatenate([zx, x[:-1]], axis=0)
        dnx = jnp.concatenate([x[1:], zx], axis=0)
        upy = jnp.concatenate([zy, x[:, :-1]], axis=1)
        dny = jnp.concatenate([x[:, 1:], zy], axis=1)
        upz = jnp.concatenate([zz, x[:, :, :-1]], axis=2)
        dnz = jnp.concatenate([x[:, :, 1:], zz], axis=2)
        v = upx + dnx + upy + dny + upz + dnz - 6.0 * x
        out_ref[:, :, :] = v

        @pl.when(mx == 0)
        def _():
            out_ref[0, :, :] = jnp.zeros((n1, n2), x.dtype)

        @pl.when(mx == SX - 1)
        def _():
            out_ref[n0 - 1, :, :] = jnp.zeros((n1, n2), x.dtype)

        @pl.when(my == 0)
        def _():
            out_ref[:, 0, :] = jnp.zeros((n0, n2), x.dtype)

        @pl.when(my == SY - 1)
        def _():
            out_ref[:, n1 - 1, :] = jnp.zeros((n0, n2), x.dtype)

        @pl.when(mz == 0)
        def _():
            out_ref[:, :, 0] = jnp.zeros((n0, n1), x.dtype)

        @pl.when(mz == SZ - 1)
        def _():
            out_ref[:, :, n2 - 1] = jnp.zeros((n0, n1), x.dtype)

        pl.semaphore_wait(barrier_sem, n_nbrs)

        for d in range(6):
            axis, plus = d // 2, d % 2
            delta = 1 if plus else -1
            dst_idx = 2 * axis + (0 if plus else 1)

            @pl.when(has[d])
            def _(d=d, axis=axis, delta=delta, dst_idx=dst_idx):
                rdma = pltpu.make_async_remote_copy(
                    src_ref=stage[d],
                    dst_ref=halos[dst_idx],
                    send_sem=send_sems.at[d],
                    recv_sem=recv_sems.at[dst_idx],
                    device_id=nbr_id(axis, delta),
                    device_id_type=pl.DeviceIdType.MESH,
                )
                rdma.start()

        for h in range(6):

            @pl.when(has[h])
            def _(h=h):
                rdma = pltpu.make_async_remote_copy(
                    src_ref=stage[h],
                    dst_ref=halos[h],
                    send_sem=send_sems.at[h],
                    recv_sem=recv_sems.at[h],
                    device_id=(mx, my, mz),
                    device_id_type=pl.DeviceIdType.MESH,
                )
                rdma.wait_recv()

        a2 = lax.broadcasted_iota(jnp.int32, (n1, n2), 0)
        b2 = lax.broadcasted_iota(jnp.int32, (n1, n2), 1)
        mask_jk = (
            ((my == 0) & (a2 == 0)) | ((my == SY - 1) & (a2 == n1 - 1))
            | ((mz == 0) & (b2 == 0)) | ((mz == SZ - 1) & (b2 == n2 - 1))
        )
        mask_ik = (
            ((mx == 0) & (a2 == 0)) | ((mx == SX - 1) & (a2 == n0 - 1))
            | ((mz == 0) & (b2 == 0)) | ((mz == SZ - 1) & (b2 == n2 - 1))
        )
        mask_ij = (
            ((mx == 0) & (a2 == 0)) | ((mx == SX - 1) & (a2 == n0 - 1))
            | ((my == 0) & (b2 == 0)) | ((my == SY - 1) & (b2 == n1 - 1))
        )

        @pl.when(has[0])
        def _():
            out_ref[0, :, :] = out_ref[0, :, :] + jnp.where(
                mask_jk, 0.0, hxm[0, :, :])

        @pl.when(has[1])
        def _():
            out_ref[n0 - 1, :, :] = out_ref[n0 - 1, :, :] + jnp.where(
                mask_jk, 0.0, hxp[0, :, :])

        @pl.when(has[2])
        def _():
            out_ref[:, 0, :] = out_ref[:, 0, :] + jnp.where(
                mask_ik, 0.0, hym[0, :, :])

        @pl.when(has[3])
        def _():
            out_ref[:, n1 - 1, :] = out_ref[:, n1 - 1, :] + jnp.where(
                mask_ik, 0.0, hyp[0, :, :])

        @pl.when(has[4])
        def _():
            out_ref[:, :, 0] = out_ref[:, :, 0] + jnp.where(
                mask_ij, 0.0, hzm[0, :, :])

        @pl.when(has[5])
        def _():
            out_ref[:, :, n2 - 1] = out_ref[:, :, n2 - 1] + jnp.where(
                mask_ij, 0.0, hzp[0, :, :])

        for d in range(6):
            axis, plus = d // 2, d % 2
            delta = 1 if plus else -1
            dst_idx = 2 * axis + (0 if plus else 1)

            @pl.when(has[d])
            def _(d=d, axis=axis, delta=delta, dst_idx=dst_idx):
                rdma = pltpu.make_async_remote_copy(
                    src_ref=stage[d],
                    dst_ref=halos[dst_idx],
                    send_sem=send_sems.at[d],
                    recv_sem=recv_sems.at[dst_idx],
                    device_id=nbr_id(axis, delta),
                    device_id_type=pl.DeviceIdType.MESH,
                )
                rdma.wait_send()

    planes = [
        pltpu.VMEM((1, n1, n2), u.dtype),
        pltpu.VMEM((1, n1, n2), u.dtype),
        pltpu.VMEM((1, n0, n2), u.dtype),
        pltpu.VMEM((1, n0, n2), u.dtype),
        pltpu.VMEM((1, n0, n1), u.dtype),
        pltpu.VMEM((1, n0, n1), u.dtype),
    ]
    return pl.pallas_call(
        body,
        out_shape=jax.ShapeDtypeStruct((n0, n1, n2), u.dtype),
        in_specs=[pl.BlockSpec(memory_space=pltpu.VMEM)],
        out_specs=pl.BlockSpec(memory_space=pltpu.VMEM),
        scratch_shapes=planes + planes + [
            pltpu.SemaphoreType.DMA((6,)),
            pltpu.SemaphoreType.DMA((6,)),
        ],
        compiler_params=pltpu.CompilerParams(collective_id=0),
    )(u)
